# baseline (speedup 1.0000x reference)
import jax
import jax.numpy as jnp
from jax import lax
from jax.experimental import pallas as pl
from jax.experimental.pallas import tpu as pltpu
from jax.experimental.pallas import tpu_sc as plsc

_SEQ = 8192
_DIM = 128
_HID = 256
_VPAD = 128
_NC = 1
_NS = 16
_NW = _NC * _NS
_LANES = 16
_SC_SEQ = 4096
_TC_SEQ = _SEQ - _SC_SEQ
_CHUNK = _SC_SEQ // _NW


def _sc_hist_body(ids_hbm, out_hbm, idsv, counts_v, sem):
    c = lax.axis_index("c")
    s = lax.axis_index("s")
    wid = c * _NS + s

    cp = pltpu.make_async_copy(ids_hbm.at[pl.ds(wid * _CHUNK, _CHUNK)], idsv, sem)
    cp.start()

    zeros = jnp.zeros((_LANES,), jnp.float32)

    def _zero(j, carry):
        counts_v[pl.ds(pl.multiple_of(j * _LANES, _LANES), _LANES)] = zeros
        return carry

    lax.fori_loop(0, _VPAD // _LANES, _zero, 0, unroll=False)
    cp.wait()

    ones = jnp.ones((_LANES,), jnp.float32)

    def _accum(k, carry):
        idvec = idsv[pl.ds(pl.multiple_of(k * _LANES, _LANES), _LANES)]
        plsc.addupdate_scatter(counts_v, [idvec], ones)
        return carry

    lax.fori_loop(0, _CHUNK // _LANES, _accum, 0, unroll=False)

    pltpu.sync_copy(counts_v, out_hbm.at[wid])


@jax.jit
def _sc_hist(ids_sc):
    mesh = plsc.VectorSubcoreMesh(
        core_axis_name="c", subcore_axis_name="s",
        num_cores=_NC, num_subcores=_NS)
    return pl.kernel(
        _sc_hist_body,
        out_type=jax.ShapeDtypeStruct((_NW, _VPAD), jnp.float32),
        mesh=mesh,
        scratch_types=[
            pltpu.VMEM((_CHUNK,), jnp.int32),
            pltpu.VMEM((_VPAD,), jnp.float32),
            pltpu.SemaphoreType.DMA,
        ],
        compiler_params=pltpu.CompilerParams(needs_layout_passes=False),
    )(ids_sc)


def _tc_hist_body(ids_ref, out_ref):
    ids = ids_ref[...]
    iota = jax.lax.broadcasted_iota(jnp.int32, (_TC_SEQ, _VPAD), 1)
    onehot = (ids == iota).astype(jnp.float32)
    out_ref[...] = jnp.sum(onehot, axis=0, keepdims=True)


def _tc_mlp_body(ca_ref, cs_ref, tab_ref, w1_ref, b1_ref, w2_ref, b2_ref,
                 out_ref):
    counts = (ca_ref[...] +
              jnp.sum(cs_ref[...], axis=0, keepdims=True))
    avg = jnp.dot(counts[:, : tab_ref.shape[0]], tab_ref[...],
                  preferred_element_type=jnp.float32,
                  precision=lax.Precision.HIGHEST) * (1.0 / _SEQ)
    h = jnp.maximum(
        jnp.dot(avg, w1_ref[...], preferred_element_type=jnp.float32,
                precision=lax.Precision.HIGHEST) + b1_ref[...], 0.0)
    out_ref[...] = jnp.dot(
        h, w2_ref[...], preferred_element_type=jnp.float32,
        precision=lax.Precision.HIGHEST) + b2_ref[...]


def kernel(ids, emb_table, W1, b1, W2, b2):
    counts_sc = _sc_hist(ids[_TC_SEQ:])
    counts_tc = pl.pallas_call(
        _tc_hist_body,
        out_shape=jax.ShapeDtypeStruct((1, _VPAD), jnp.float32),
    )(ids[:_TC_SEQ].reshape(_TC_SEQ, 1))
    out = pl.pallas_call(
        _tc_mlp_body,
        out_shape=jax.ShapeDtypeStruct((1, _HID), jnp.float32),
    )(counts_tc, counts_sc, emb_table, W1, b1.reshape(1, _HID),
      W2, b2.reshape(1, _HID))
    return out.reshape(_HID)


# speedup vs baseline: 1.0503x; 1.0503x over previous
import jax
import jax.numpy as jnp
from jax import lax
from jax.experimental import pallas as pl
from jax.experimental.pallas import tpu as pltpu
from jax.experimental.pallas import tpu_sc as plsc

_SEQ = 8192
_DIM = 128
_HID = 256
_VPAD = 128
_NC = 1
_NS = 16
_NW = _NC * _NS
_LANES = 16
_CHUNK = _SEQ // _NW


def _sc_hist_body(ids_hbm, out_hbm, idsv, counts_v, sem):
    c = lax.axis_index("c")
    s = lax.axis_index("s")
    wid = c * _NS + s

    cp = pltpu.make_async_copy(ids_hbm.at[pl.ds(wid * _CHUNK, _CHUNK)], idsv, sem)
    cp.start()

    zeros = jnp.zeros((_LANES,), jnp.float32)

    def _zero(j, carry):
        counts_v[pl.ds(pl.multiple_of(j * _LANES, _LANES), _LANES)] = zeros
        return carry

    lax.fori_loop(0, _VPAD // _LANES, _zero, 0, unroll=False)
    cp.wait()

    ones = jnp.ones((_LANES,), jnp.float32)

    def _accum(k, carry):
        idvec = idsv[pl.ds(pl.multiple_of(k * _LANES, _LANES), _LANES)]
        plsc.addupdate_scatter(counts_v, [idvec], ones)
        return carry

    lax.fori_loop(0, _CHUNK // _LANES, _accum, 0, unroll=4)

    pltpu.sync_copy(counts_v, out_hbm.at[wid])


@jax.jit
def _sc_hist(ids):
    mesh = plsc.VectorSubcoreMesh(
        core_axis_name="c", subcore_axis_name="s",
        num_cores=_NC, num_subcores=_NS)
    return pl.kernel(
        _sc_hist_body,
        out_type=jax.ShapeDtypeStruct((_NW, _VPAD), jnp.float32),
        mesh=mesh,
        scratch_types=[
            pltpu.VMEM((_CHUNK,), jnp.int32),
            pltpu.VMEM((_VPAD,), jnp.float32),
            pltpu.SemaphoreType.DMA,
        ],
        compiler_params=pltpu.CompilerParams(needs_layout_passes=False),
    )(ids)


def _tc_mlp_body(cnt_ref, tab_ref, w1_ref, b1_ref, w2_ref, b2_ref, out_ref):
    c2 = cnt_ref[...]
    counts = jnp.sum(c2[:, : tab_ref.shape[0]], axis=0, keepdims=True)
    avg = jnp.dot(counts, tab_ref[...],
                  preferred_element_type=jnp.float32,
                  precision=lax.Precision.DEFAULT) * (1.0 / _SEQ)
    h = jnp.maximum(
        jnp.dot(avg, w1_ref[...], preferred_element_type=jnp.float32,
                precision=lax.Precision.DEFAULT) + b1_ref[...], 0.0)
    out_ref[...] = jnp.dot(
        h, w2_ref[...], preferred_element_type=jnp.float32,
        precision=lax.Precision.DEFAULT) + b2_ref[...]


def kernel(ids, emb_table, W1, b1, W2, b2):
    counts16 = _sc_hist(ids)
    out = pl.pallas_call(
        _tc_mlp_body,
        out_shape=jax.ShapeDtypeStruct((1, _HID), jnp.float32),
    )(counts16, emb_table, W1, b1.reshape(1, _HID), W2, b2.reshape(1, _HID))
    return out.reshape(_HID)
